# baseline (speedup 1.0000x reference)
import functools

import jax
import jax.numpy as jnp
from jax import lax
from jax.experimental import pallas as pl
from jax.experimental.pallas import tpu as pltpu
from jax.experimental.pallas import tpu_sc as plsc

N = 10000
NP = 10240
E = 320000
D = 128
NCLS = 16
NC, NS = 2, 16
NW = NC * NS
EP = 327680
EDGES_PER_W = EP // NW
ROWS_PER_W = EDGES_PER_W // 128
CH0 = 80
CH1 = 160 - CH0
SB_ROWS = 16
SLAB = NP // NS
TRASH = N + 100
BM = 512


def _sc_mesh():
    return plsc.VectorSubcoreMesh(core_axis_name="c", subcore_axis_name="s")


@functools.partial(
    pl.kernel,
    out_type=jax.ShapeDtypeStruct((NC, NP), jnp.float32),
    mesh=_sc_mesh(),
    scratch_types=[
        pltpu.VMEM((8, 128), jnp.int32),
        pltpu.VMEM((128,), jnp.float32),
        pltpu.VMEM((SLAB,), jnp.float32),
        pltpu.VMEM_SHARED((NP,), jnp.float32),
    ],
)
def _deg_kernel(dst3_hbm, ones_hbm, zeros_hbm, out_hbm, didx_v, ones_v,
                buf_v, deg_sh):
    c = lax.axis_index("c")
    s = lax.axis_index("s")
    w = c * NS + s
    pltpu.sync_copy(zeros_hbm, buf_v)
    pltpu.sync_copy(buf_v, deg_sh.at[pl.ds(s * SLAB, SLAB)])
    pltpu.sync_copy(ones_hbm, ones_v)
    plsc.subcore_barrier()

    def outer(jj, carry):
        pltpu.sync_copy(dst3_hbm.at[pl.ds(w * ROWS_PER_W + jj * 8, 8)],
                        didx_v)

        def body(k, c2):
            pltpu.sync_copy(ones_v, deg_sh.at[didx_v.at[k]], add=True)
            return c2

        lax.fori_loop(0, 8, body, carry)
        return carry

    lax.fori_loop(0, ROWS_PER_W // 8, outer, 0)
    plsc.subcore_barrier()
    pltpu.sync_copy(deg_sh.at[pl.ds(s * SLAB, SLAB)],
                    out_hbm.at[c, pl.ds(s * SLAB, SLAB)])


def _make_gather_sum(F):
    @functools.partial(
        pl.kernel,
        out_type=jax.ShapeDtypeStruct((NC, NP, F), jnp.float32),
        mesh=_sc_mesh(),
        scratch_types=[
            pltpu.VMEM((2 * SB_ROWS, 128), jnp.int32),
            pltpu.VMEM((128, F), jnp.float32),
            pltpu.VMEM((128, F), jnp.float32),
            pltpu.VMEM_SHARED((NP, F), jnp.float32),
            pltpu.SemaphoreType.DMA,
            pltpu.SemaphoreType.DMA,
        ],
    )
    def gs(z_hbm, src3_hbm, dst3_hbm, out_hbm,
           idx_v, rows_a, rows_b, acc_sh, sem_a, sem_b):
        c = lax.axis_index("c")
        s = lax.axis_index("s")
        w = c * NS + s

        def zbody(i, carry):
            for jj in range(F // 16):
                rows_a[i, pl.ds(16 * jj, 16)] = jnp.zeros((16,), jnp.float32)
            return carry

        lax.fori_loop(0, 128, zbody, 0)
        for i in range(SLAB // 128):
            pltpu.sync_copy(rows_a,
                            acc_sh.at[pl.ds(s * SLAB + i * 128, 128)])
        plsc.subcore_barrier()

        def run(chunk_base, n_sb):
            for sb in range(n_sb):
                base = chunk_base + sb * SB_ROWS
                pltpu.sync_copy(src3_hbm.at[pl.ds(base, SB_ROWS)],
                                idx_v.at[pl.ds(0, SB_ROWS)])
                pltpu.sync_copy(dst3_hbm.at[pl.ds(base, SB_ROWS)],
                                idx_v.at[pl.ds(SB_ROWS, SB_ROWS)])
                pltpu.async_copy(z_hbm.at[idx_v.at[0]], rows_a, sem_a)

                def body(i, carry):
                    j = 2 * i
                    pltpu.make_async_copy(z_hbm.at[idx_v.at[j]], rows_a,
                                          sem_a).wait()
                    pltpu.async_copy(z_hbm.at[idx_v.at[j + 1]], rows_b,
                                     sem_b)
                    pltpu.sync_copy(rows_a, acc_sh.at[idx_v.at[SB_ROWS + j]],
                                    add=True)
                    pltpu.make_async_copy(z_hbm.at[idx_v.at[j + 1]], rows_b,
                                          sem_b).wait()

                    @pl.when(j + 2 < SB_ROWS)
                    def _():
                        pltpu.async_copy(z_hbm.at[idx_v.at[j + 2]], rows_a,
                                         sem_a)

                    pltpu.sync_copy(rows_b,
                                    acc_sh.at[idx_v.at[SB_ROWS + j + 1]],
                                    add=True)
                    return carry

                lax.fori_loop(0, SB_ROWS // 2, body, 0)

        @pl.when(c == 0)
        def _():
            run(s * CH0, CH0 // SB_ROWS)

        @pl.when(c == 1)
        def _():
            run(NS * CH0 + s * CH1, CH1 // SB_ROWS)

        plsc.subcore_barrier()
        pltpu.sync_copy(acc_sh.at[pl.ds(s * SLAB, SLAB)],
                        out_hbm.at[c, pl.ds(s * SLAB, SLAB)])

    return gs


_gs128 = _make_gather_sum(D)


def _mm_body(x_ref, w_ref, h_ref):
    h_ref[...] = jnp.dot(x_ref[...], w_ref[...],
                         preferred_element_type=jnp.float32)


def _tc_mm(x_pad, W):
    return pl.pallas_call(
        _mm_body,
        grid=(NP // BM,),
        in_specs=[
            pl.BlockSpec((BM, D), lambda i: (i, 0)),
            pl.BlockSpec((D, D), lambda i: (0, 0)),
        ],
        out_specs=pl.BlockSpec((BM, D), lambda i: (i, 0)),
        out_shape=jax.ShapeDtypeStruct((NP, D), jnp.float32),
    )(x_pad, W)


def _scale_body(h_ref, dega_ref, degb_ref, z_ref):
    dinv = lax.rsqrt(dega_ref[...] + degb_ref[...] + 1.0)
    z_ref[...] = dinv * h_ref[...]


def _tc_scale(h, dega, degb):
    return pl.pallas_call(
        _scale_body,
        grid=(NP // BM,),
        in_specs=[
            pl.BlockSpec((BM, D), lambda i: (i, 0)),
            pl.BlockSpec((BM, 1), lambda i: (i, 0)),
            pl.BlockSpec((BM, 1), lambda i: (i, 0)),
        ],
        out_specs=pl.BlockSpec((BM, D), lambda i: (i, 0)),
        out_shape=jax.ShapeDtypeStruct((NP, D), jnp.float32),
    )(h, dega, degb)


def _combine_body(acc_ref, z_ref, dega_ref, degb_ref, b_ref, w_ref, out_ref):
    dinv = lax.rsqrt(dega_ref[...] + degb_ref[...] + 1.0)
    t = dinv * (acc_ref[0] + acc_ref[1] + z_ref[...]) + b_ref[...]
    t = jnp.maximum(t, 0.0)
    out_ref[...] = dinv * jnp.dot(t, w_ref[...],
                                  preferred_element_type=jnp.float32)


def _tc_combine(acc, z_prev, dega, degb, b, W):
    dout = W.shape[1]
    return pl.pallas_call(
        _combine_body,
        grid=(NP // BM,),
        in_specs=[
            pl.BlockSpec((NC, BM, D), lambda i: (0, i, 0)),
            pl.BlockSpec((BM, D), lambda i: (i, 0)),
            pl.BlockSpec((BM, 1), lambda i: (i, 0)),
            pl.BlockSpec((BM, 1), lambda i: (i, 0)),
            pl.BlockSpec((1, D), lambda i: (0, 0)),
            pl.BlockSpec((D, dout), lambda i: (0, 0)),
        ],
        out_specs=pl.BlockSpec((BM, dout), lambda i: (i, 0)),
        out_shape=jax.ShapeDtypeStruct((NP, dout), jnp.float32),
    )(acc, z_prev, dega, degb, b, W)


def _relu_scale_body(acc_ref, z_ref, dega_ref, degb_ref, b_ref, out_ref):
    dinv = lax.rsqrt(dega_ref[...] + degb_ref[...] + 1.0)
    t = dinv * (acc_ref[0] + acc_ref[1] + z_ref[...]) + b_ref[...]
    out_ref[...] = dinv * jnp.maximum(t, 0.0)


def _tc_relu_scale(acc, z_prev, dega, degb, b):
    return pl.pallas_call(
        _relu_scale_body,
        grid=(NP // BM,),
        in_specs=[
            pl.BlockSpec((NC, BM, D), lambda i: (0, i, 0)),
            pl.BlockSpec((BM, D), lambda i: (i, 0)),
            pl.BlockSpec((BM, 1), lambda i: (i, 0)),
            pl.BlockSpec((BM, 1), lambda i: (i, 0)),
            pl.BlockSpec((1, D), lambda i: (0, 0)),
        ],
        out_specs=pl.BlockSpec((BM, D), lambda i: (i, 0)),
        out_shape=jax.ShapeDtypeStruct((NP, D), jnp.float32),
    )(acc, z_prev, dega, degb, b)


def _final_body(acc_ref, y_ref, dega_ref, degb_ref, b_ref, w_ref, out_ref):
    dinv = lax.rsqrt(dega_ref[...] + degb_ref[...] + 1.0)
    u = dinv * (acc_ref[0] + acc_ref[1] + y_ref[...])
    logits = jnp.dot(u, w_ref[...],
                     preferred_element_type=jnp.float32) + b_ref[...]
    m = jnp.max(logits, axis=1, keepdims=True)
    lse = jnp.log(jnp.sum(jnp.exp(logits - m), axis=1, keepdims=True)) + m
    out_ref[...] = logits - lse


def _tc_final(acc, y2, dega, degb, b, W):
    return pl.pallas_call(
        _final_body,
        grid=(NP // BM,),
        in_specs=[
            pl.BlockSpec((NC, BM, D), lambda i: (0, i, 0)),
            pl.BlockSpec((BM, D), lambda i: (i, 0)),
            pl.BlockSpec((BM, 1), lambda i: (i, 0)),
            pl.BlockSpec((BM, 1), lambda i: (i, 0)),
            pl.BlockSpec((1, NCLS), lambda i: (0, 0)),
            pl.BlockSpec((D, NCLS), lambda i: (0, 0)),
        ],
        out_specs=pl.BlockSpec((BM, NCLS), lambda i: (i, 0)),
        out_shape=jax.ShapeDtypeStruct((NP, NCLS), jnp.float32),
    )(acc, y2, dega, degb, b, W)


def kernel(x, edge_index, W0, b0, W1, b1, W2, b2):
    src = edge_index[0].astype(jnp.int32)
    dst = edge_index[1].astype(jnp.int32)
    pad_iota = jnp.arange(EP - E, dtype=jnp.int32)
    src_pad = jnp.concatenate([src, (pad_iota * 997) % N])
    pad_dst = N + pad_iota % (NP - N)
    dst_pad = jnp.concatenate([dst, pad_dst])
    src3 = src_pad.reshape(EP // 128, 128)
    dst3 = dst_pad.reshape(EP // 128, 128)
    x_pad = jnp.pad(x, ((0, NP - N), (0, 0)))

    ones128 = jnp.ones((128,), jnp.float32)
    zeros_slab = jnp.zeros((SLAB,), jnp.float32)

    h0 = _tc_mm(x_pad, W0)
    deg2 = _deg_kernel(dst3, ones128, zeros_slab)
    dega = deg2[0].reshape(NP, 1)
    degb = deg2[1].reshape(NP, 1)

    z0 = _tc_scale(h0, dega, degb)
    acc0 = _gs128(z0, src3, dst3)
    z1 = _tc_combine(acc0, z0, dega, degb, b0.reshape(1, D), W1)
    acc1 = _gs128(z1, src3, dst3)
    y2 = _tc_relu_scale(acc1, z1, dega, degb, b1.reshape(1, D))
    acc2 = _gs128(y2, src3, dst3)
    out = _tc_final(acc2, y2, dega, degb, b2.reshape(1, NCLS), W2)
    return out[:N]


# speedup vs baseline: 1.0037x; 1.0037x over previous
import functools

import jax
import jax.numpy as jnp
from jax import lax
from jax.experimental import pallas as pl
from jax.experimental.pallas import tpu as pltpu
from jax.experimental.pallas import tpu_sc as plsc

N = 10000
NP = 10240
E = 320000
D = 128
NCLS = 16
NC, NS = 2, 16
NW = NC * NS
EP = 327680
EDGES_PER_W = EP // NW
ROWS_PER_W = EDGES_PER_W // 128
CH0 = 80
CH1 = 160 - CH0
SB_ROWS = 16
SLAB = NP // NS
TRASH = N + 100
BM = 512


def _sc_mesh():
    return plsc.VectorSubcoreMesh(core_axis_name="c", subcore_axis_name="s")


@functools.partial(
    pl.kernel,
    out_type=jax.ShapeDtypeStruct((NC, NP), jnp.float32),
    mesh=_sc_mesh(),
    scratch_types=[
        pltpu.VMEM((8, 128), jnp.int32),
        pltpu.VMEM((128,), jnp.float32),
        pltpu.VMEM((SLAB,), jnp.float32),
        pltpu.VMEM_SHARED((NP,), jnp.float32),
    ],
)
def _deg_kernel(dst3_hbm, ones_hbm, zeros_hbm, out_hbm, didx_v, ones_v,
                buf_v, deg_sh):
    c = lax.axis_index("c")
    s = lax.axis_index("s")
    w = c * NS + s
    pltpu.sync_copy(zeros_hbm, buf_v)
    pltpu.sync_copy(buf_v, deg_sh.at[pl.ds(s * SLAB, SLAB)])
    pltpu.sync_copy(ones_hbm, ones_v)
    plsc.subcore_barrier()

    def outer(jj, carry):
        pltpu.sync_copy(dst3_hbm.at[pl.ds(w * ROWS_PER_W + jj * 8, 8)],
                        didx_v)

        def body(k, c2):
            pltpu.sync_copy(ones_v, deg_sh.at[didx_v.at[k]], add=True)
            return c2

        lax.fori_loop(0, 8, body, carry)
        return carry

    lax.fori_loop(0, ROWS_PER_W // 8, outer, 0)
    plsc.subcore_barrier()
    pltpu.sync_copy(deg_sh.at[pl.ds(s * SLAB, SLAB)],
                    out_hbm.at[c, pl.ds(s * SLAB, SLAB)])


def _make_gather_sum(F):
    @functools.partial(
        pl.kernel,
        out_type=jax.ShapeDtypeStruct((NC, NP, F), jnp.float32),
        mesh=_sc_mesh(),
        scratch_types=[
            pltpu.VMEM((2 * SB_ROWS, 128), jnp.int32),
            pltpu.VMEM((128, F), jnp.float32),
            pltpu.VMEM((128, F), jnp.float32),
            pltpu.VMEM_SHARED((NP, F), jnp.float32),
            pltpu.SemaphoreType.DMA,
            pltpu.SemaphoreType.DMA,
        ],
    )
    def gs(z_hbm, src3_hbm, dst3_hbm, out_hbm,
           idx_v, rows_a, rows_b, acc_sh, sem_a, sem_b):
        c = lax.axis_index("c")
        s = lax.axis_index("s")
        w = c * NS + s

        def zbody(i, carry):
            for jj in range(F // 16):
                rows_a[i, pl.ds(16 * jj, 16)] = jnp.zeros((16,), jnp.float32)
            return carry

        lax.fori_loop(0, 128, zbody, 0)
        for i in range(SLAB // 128):
            pltpu.sync_copy(rows_a,
                            acc_sh.at[pl.ds(s * SLAB + i * 128, 128)])
        plsc.subcore_barrier()

        def run(chunk_base, n_sb):
            for sb in range(n_sb):
                base = chunk_base + sb * SB_ROWS
                pltpu.sync_copy(src3_hbm.at[pl.ds(base, SB_ROWS)],
                                idx_v.at[pl.ds(0, SB_ROWS)])
                pltpu.sync_copy(dst3_hbm.at[pl.ds(base, SB_ROWS)],
                                idx_v.at[pl.ds(SB_ROWS, SB_ROWS)])
                pltpu.async_copy(z_hbm.at[idx_v.at[0]], rows_a, sem_a)

                def body(i, carry):
                    j = 2 * i
                    pltpu.make_async_copy(z_hbm.at[idx_v.at[j]], rows_a,
                                          sem_a).wait()
                    pltpu.async_copy(z_hbm.at[idx_v.at[j + 1]], rows_b,
                                     sem_b)
                    pltpu.sync_copy(rows_a, acc_sh.at[idx_v.at[SB_ROWS + j]],
                                    add=True)
                    pltpu.make_async_copy(z_hbm.at[idx_v.at[j + 1]], rows_b,
                                          sem_b).wait()

                    @pl.when(j + 2 < SB_ROWS)
                    def _():
                        pltpu.async_copy(z_hbm.at[idx_v.at[j + 2]], rows_a,
                                         sem_a)

                    pltpu.sync_copy(rows_b,
                                    acc_sh.at[idx_v.at[SB_ROWS + j + 1]],
                                    add=True)
                    return carry

                lax.fori_loop(0, SB_ROWS // 2, body, 0)

        @pl.when(c == 0)
        def _():
            run(s * CH0, CH0 // SB_ROWS)

        @pl.when(c == 1)
        def _():
            run(NS * CH0 + s * CH1, CH1 // SB_ROWS)

        plsc.subcore_barrier()
        pltpu.sync_copy(acc_sh.at[pl.ds(s * SLAB, SLAB)],
                        out_hbm.at[c, pl.ds(s * SLAB, SLAB)])

    return gs


_gs128 = _make_gather_sum(D)


def _tc1_body(x_ref, w_ref, dega_ref, degb_ref, z_ref):
    dinv = lax.rsqrt(dega_ref[...] + degb_ref[...] + 1.0)
    h = jnp.dot(x_ref[...], w_ref[...], preferred_element_type=jnp.float32)
    z_ref[...] = dinv * h


def _tc1(x_pad, W, dega, degb):
    return pl.pallas_call(
        _tc1_body,
        grid=(NP // BM,),
        in_specs=[
            pl.BlockSpec((BM, D), lambda i: (i, 0)),
            pl.BlockSpec((D, D), lambda i: (0, 0)),
            pl.BlockSpec((BM, 1), lambda i: (i, 0)),
            pl.BlockSpec((BM, 1), lambda i: (i, 0)),
        ],
        out_specs=pl.BlockSpec((BM, D), lambda i: (i, 0)),
        out_shape=jax.ShapeDtypeStruct((NP, D), jnp.float32),
    )(x_pad, W, dega, degb)


def _combine_body(acc_ref, z_ref, dega_ref, degb_ref, b_ref, w_ref, out_ref):
    dinv = lax.rsqrt(dega_ref[...] + degb_ref[...] + 1.0)
    t = dinv * (acc_ref[0] + acc_ref[1] + z_ref[...]) + b_ref[...]
    t = jnp.maximum(t, 0.0)
    out_ref[...] = dinv * jnp.dot(t, w_ref[...],
                                  preferred_element_type=jnp.float32)


def _tc_combine(acc, z_prev, dega, degb, b, W):
    dout = W.shape[1]
    return pl.pallas_call(
        _combine_body,
        grid=(NP // BM,),
        in_specs=[
            pl.BlockSpec((NC, BM, D), lambda i: (0, i, 0)),
            pl.BlockSpec((BM, D), lambda i: (i, 0)),
            pl.BlockSpec((BM, 1), lambda i: (i, 0)),
            pl.BlockSpec((BM, 1), lambda i: (i, 0)),
            pl.BlockSpec((1, D), lambda i: (0, 0)),
            pl.BlockSpec((D, dout), lambda i: (0, 0)),
        ],
        out_specs=pl.BlockSpec((BM, dout), lambda i: (i, 0)),
        out_shape=jax.ShapeDtypeStruct((NP, dout), jnp.float32),
    )(acc, z_prev, dega, degb, b, W)


def _relu_scale_body(acc_ref, z_ref, dega_ref, degb_ref, b_ref, out_ref):
    dinv = lax.rsqrt(dega_ref[...] + degb_ref[...] + 1.0)
    t = dinv * (acc_ref[0] + acc_ref[1] + z_ref[...]) + b_ref[...]
    out_ref[...] = dinv * jnp.maximum(t, 0.0)


def _tc_relu_scale(acc, z_prev, dega, degb, b):
    return pl.pallas_call(
        _relu_scale_body,
        grid=(NP // BM,),
        in_specs=[
            pl.BlockSpec((NC, BM, D), lambda i: (0, i, 0)),
            pl.BlockSpec((BM, D), lambda i: (i, 0)),
            pl.BlockSpec((BM, 1), lambda i: (i, 0)),
            pl.BlockSpec((BM, 1), lambda i: (i, 0)),
            pl.BlockSpec((1, D), lambda i: (0, 0)),
        ],
        out_specs=pl.BlockSpec((BM, D), lambda i: (i, 0)),
        out_shape=jax.ShapeDtypeStruct((NP, D), jnp.float32),
    )(acc, z_prev, dega, degb, b)


def _final_body(acc_ref, y_ref, dega_ref, degb_ref, b_ref, w_ref, out_ref):
    dinv = lax.rsqrt(dega_ref[...] + degb_ref[...] + 1.0)
    u = dinv * (acc_ref[0] + acc_ref[1] + y_ref[...])
    logits = jnp.dot(u, w_ref[...],
                     preferred_element_type=jnp.float32) + b_ref[...]
    m = jnp.max(logits, axis=1, keepdims=True)
    lse = jnp.log(jnp.sum(jnp.exp(logits - m), axis=1, keepdims=True)) + m
    out_ref[...] = logits - lse


def _tc_final(acc, y2, dega, degb, b, W):
    return pl.pallas_call(
        _final_body,
        grid=(NP // BM,),
        in_specs=[
            pl.BlockSpec((NC, BM, D), lambda i: (0, i, 0)),
            pl.BlockSpec((BM, D), lambda i: (i, 0)),
            pl.BlockSpec((BM, 1), lambda i: (i, 0)),
            pl.BlockSpec((BM, 1), lambda i: (i, 0)),
            pl.BlockSpec((1, NCLS), lambda i: (0, 0)),
            pl.BlockSpec((D, NCLS), lambda i: (0, 0)),
        ],
        out_specs=pl.BlockSpec((BM, NCLS), lambda i: (i, 0)),
        out_shape=jax.ShapeDtypeStruct((NP, NCLS), jnp.float32),
    )(acc, y2, dega, degb, b, W)


def kernel(x, edge_index, W0, b0, W1, b1, W2, b2):
    src = edge_index[0].astype(jnp.int32)
    dst = edge_index[1].astype(jnp.int32)
    pad_iota = jnp.arange(EP - E, dtype=jnp.int32)
    src_pad = jnp.concatenate([src, (pad_iota * 997) % N])
    pad_dst = N + pad_iota % (NP - N)
    dst_pad = jnp.concatenate([dst, pad_dst])
    src3 = src_pad.reshape(EP // 128, 128)
    dst3 = dst_pad.reshape(EP // 128, 128)
    x_pad = jnp.pad(x, ((0, NP - N), (0, 0)))

    ones128 = jnp.ones((128,), jnp.float32)
    zeros_slab = jnp.zeros((SLAB,), jnp.float32)

    deg2 = _deg_kernel(dst3, ones128, zeros_slab)
    dega = deg2[0].reshape(NP, 1)
    degb = deg2[1].reshape(NP, 1)

    z0 = _tc1(x_pad, W0, dega, degb)
    acc0 = _gs128(z0, src3, dst3)
    z1 = _tc_combine(acc0, z0, dega, degb, b0.reshape(1, D), W1)
    acc1 = _gs128(z1, src3, dst3)
    y2 = _tc_relu_scale(acc1, z1, dega, degb, b1.reshape(1, D))
    acc2 = _gs128(y2, src3, dst3)
    out = _tc_final(acc2, y2, dega, degb, b2.reshape(1, NCLS), W2)
    return out[:N]
